# baseline (speedup 1.0000x reference)
import jax
import jax.numpy as jnp
from jax import lax
from jax.experimental import pallas as pl
from jax.experimental.pallas import tpu as pltpu
from jax.experimental.pallas import tpu_sc as plsc

VOCAB = 1000000
BINS = 100
D = 64
B = 4096
H = 200
NC, NS = 2, 16
NW = NC * NS
BPW = B // NW
NB = 2
NV = BPW // 16


def _body(idsT, sidsT, wtab, stab, out, widx_v, sidx_v, stab_v,
          rows0_v, rows1_v, obuf0_v, obuf1_v, gsem, ssem):
    rows = (rows0_v, rows1_v)
    obuf = (obuf0_v, obuf1_v)
    wid = lax.axis_index("s") * NC + lax.axis_index("c")
    b0 = wid * BPW
    pltpu.sync_copy(idsT.at[:, pl.ds(b0, BPW)], widx_v)
    pltpu.sync_copy(sidsT.at[:, pl.ds(b0, BPW)], sidx_v)
    pltpu.sync_copy(stab, stab_v)

    def gather_desc(h, j):
        return pltpu.make_async_copy(
            wtab.at[widx_v.at[h]], rows[j], gsem.at[j])

    def scatter_desc(h, j):
        return pltpu.make_async_copy(
            obuf[j], out.at[h, :, pl.ds(b0, BPW)], ssem.at[j])

    for j in range(NB):
        gather_desc(j, j).start()

    @pl.loop(0, H, step=NB)
    def _h0(h0):
        for j in range(NB):
            h = h0 + j
            gather_desc(h, j).wait()

            @pl.when(h0 >= NB)
            def _():
                scatter_desc(h - NB, j).wait()

            for vb in range(NV):
                rvec = lax.iota(jnp.int32, 16) + 16 * vb
                svec = sidx_v[h, pl.ds(vb * 16, 16)] * D
                sl = pl.ds(vb * 16, 16)

                @pl.loop(0, D, unroll=8)
                def _d(d):
                    dvec = jnp.full((16,), d, jnp.int32)
                    w = plsc.load_gather(rows[j], [rvec, dvec])
                    s = plsc.load_gather(stab_v, [svec + d])
                    obuf[j][d, sl] = w + s

            scatter_desc(h, j).start()

            @pl.when(h + NB < H)
            def _():
                gather_desc(h + NB, j).start()

    for j in range(NB):
        scatter_desc(H - NB + j, j).wait()


@jax.jit
def _sc_embed(idsT, sidsT, wtab, stab):
    kern = pl.kernel(
        _body,
        out_type=jax.ShapeDtypeStruct((H, D, B), jnp.float32),
        mesh=plsc.VectorSubcoreMesh(core_axis_name="c", subcore_axis_name="s"),
        compiler_params=pltpu.CompilerParams(use_tc_tiling_on_sc=False,
                                             needs_layout_passes=False),
        scratch_types=[
            pltpu.VMEM((H, BPW), jnp.int32),
            pltpu.VMEM((H, BPW), jnp.int32),
            pltpu.VMEM((BINS * D,), jnp.float32),
            pltpu.VMEM((BPW, D), jnp.float32),
            pltpu.VMEM((BPW, D), jnp.float32),
            pltpu.VMEM((D, BPW), jnp.float32),
            pltpu.VMEM((D, BPW), jnp.float32),
            pltpu.SemaphoreType.DMA((NB,)),
            pltpu.SemaphoreType.DMA((NB,)),
        ],
    )
    return kern(idsT, sidsT, wtab, stab)


def kernel(input_ids, scores_ids, words_emb, scores_emb):
    out_t = _sc_embed(input_ids.T.astype(jnp.int32),
                      scores_ids.T.astype(jnp.int32),
                      words_emb, scores_emb.reshape(-1))
    return jnp.transpose(out_t, (2, 0, 1))


# speedup vs baseline: 1.7018x; 1.7018x over previous
import jax
import jax.numpy as jnp
from jax import lax
from jax.experimental import pallas as pl
from jax.experimental.pallas import tpu as pltpu
from jax.experimental.pallas import tpu_sc as plsc

VOCAB = 1000000
BINS = 100
D = 64
B = 4096
H = 200
NC, NS = 2, 16
NW = NC * NS
RPW = B // NW
G0, G1 = 104, 96
NB = 4


def _body(wids, sids, wtab, stab, out, widx_v, sidx_v,
          rows0, rows1, rows2, rows3, obuf0, obuf1, obuf2, obuf3,
          gsemw, gsems, ssem):
    rows = (rows0, rows1, rows2, rows3)
    obuf = (obuf0, obuf1, obuf2, obuf3)
    gs = (G0, G1, G0, G1)
    offs = (0, G0, 0, G0)
    wid = lax.axis_index("s") * NC + lax.axis_index("c")
    row0 = wid * RPW
    pltpu.sync_copy(wids.at[pl.ds(row0, RPW)], widx_v)
    pltpu.sync_copy(sids.at[pl.ds(row0, RPW)], sidx_v)

    def start_gathers(i, s):
        g, off = gs[s], offs[s]
        pltpu.make_async_copy(
            wtab.at[widx_v.at[i, pl.ds(off, g)]], rows[s],
            gsemw.at[s]).start()
        pltpu.make_async_copy(
            stab.at[sidx_v.at[i, pl.ds(off, g)]], obuf[s],
            gsems.at[s]).start()

    def wait_gathers(i, s):
        g, off = gs[s], offs[s]
        pltpu.make_async_copy(
            wtab.at[widx_v.at[i, pl.ds(off, g)]], rows[s],
            gsemw.at[s]).wait()
        pltpu.make_async_copy(
            stab.at[sidx_v.at[i, pl.ds(off, g)]], obuf[s],
            gsems.at[s]).wait()

    def scatter_desc(i, s):
        g, off = gs[s], offs[s]
        return pltpu.make_async_copy(
            obuf[s], out.at[row0 + i, pl.ds(off, g)], ssem.at[s])

    for s in range(2):
        start_gathers(0, s)

    @pl.loop(0, RPW, step=2)
    def _rowpair(i0):
        for ii in range(2):
            for j in range(2):
                i = i0 + ii
                s = 2 * ii + j

                wait_gathers(i, s)

                @pl.loop(0, gs[s], unroll=16)
                def _lk(r):
                    for c in range(D // 16):
                        sl = pl.ds(c * 16, 16)
                        obuf[s][r, sl] = obuf[s][r, sl] + rows[s][r, sl]

                scatter_desc(i, s).start()

                sp = (s + 2) % NB

                @pl.when(i + 1 < RPW)
                def _():
                    if ii == 1:
                        scatter_desc(i - 1, sp).wait()
                    else:
                        @pl.when(i0 >= 2)
                        def _():
                            scatter_desc(i - 1, sp).wait()
                    start_gathers(i + 1, sp)

    for j in range(2):
        scatter_desc(RPW - 2, j).wait()
    for s in range(2, 4):
        scatter_desc(RPW - 1, s).wait()


@jax.jit
def _sc_embed(wids, sids, wtab, stab):
    kern = pl.kernel(
        _body,
        out_type=jax.ShapeDtypeStruct((B, H, D), jnp.float32),
        mesh=plsc.VectorSubcoreMesh(core_axis_name="c", subcore_axis_name="s"),
        compiler_params=pltpu.CompilerParams(use_tc_tiling_on_sc=False,
                                             needs_layout_passes=False),
        scratch_types=[
            pltpu.VMEM((RPW, H), jnp.int32),
            pltpu.VMEM((RPW, H), jnp.int32),
            pltpu.VMEM((G0, D), jnp.float32),
            pltpu.VMEM((G1, D), jnp.float32),
            pltpu.VMEM((G0, D), jnp.float32),
            pltpu.VMEM((G1, D), jnp.float32),
            pltpu.VMEM((G0, D), jnp.float32),
            pltpu.VMEM((G1, D), jnp.float32),
            pltpu.VMEM((G0, D), jnp.float32),
            pltpu.VMEM((G1, D), jnp.float32),
            pltpu.SemaphoreType.DMA((NB,)),
            pltpu.SemaphoreType.DMA((NB,)),
            pltpu.SemaphoreType.DMA((NB,)),
        ],
    )
    return kern(wids, sids, wtab, stab)


def kernel(input_ids, scores_ids, words_emb, scores_emb):
    return _sc_embed(input_ids.astype(jnp.int32), scores_ids.astype(jnp.int32),
                     words_emb, scores_emb)


# speedup vs baseline: 1.9371x; 1.1383x over previous
import jax
import jax.numpy as jnp
from jax import lax
from jax.experimental import pallas as pl
from jax.experimental.pallas import tpu as pltpu
from jax.experimental.pallas import tpu_sc as plsc

VOCAB = 1000000
BINS = 100
D = 64
N = 4096 * 200
NC, NS = 2, 16
NW = NC * NS
PER_W = N // NW
G = 128
NG = PER_W // G
NB = 4


def _body(wids, sids, wtab, stab, out, widx_v, sidx_v, stab_v,
          rows0, rows1, rows2, rows3, obuf0, obuf1, obuf2, obuf3,
          gsem, ssem):
    rows = (rows0, rows1, rows2, rows3)
    obuf = (obuf0, obuf1, obuf2, obuf3)
    wid = lax.axis_index("s") * NC + lax.axis_index("c")
    base0 = wid * PER_W
    pltpu.sync_copy(wids.at[wid], widx_v)
    pltpu.sync_copy(sids.at[wid], sidx_v)
    pltpu.sync_copy(stab, stab_v)

    def gather_desc(g, s):
        return pltpu.make_async_copy(
            wtab.at[widx_v.at[g]], rows[s], gsem.at[s])

    def scatter_desc(g, s):
        return pltpu.make_async_copy(
            obuf[s], out.at[pl.ds(base0 + g * G, G)], ssem.at[s])

    for s in range(2):
        gather_desc(s, s).start()

    @pl.loop(0, NG, step=NB)
    def _g0(g0):
        for s in range(NB):
            g = g0 + s
            gather_desc(g, s).wait()

            @pl.loop(0, G // 16, unroll=2)
            def _chunk(c):
                svec = sidx_v[g, pl.ds(c * 16, 16)]
                r0 = c * 16
                for k in range(16):
                    sid = svec[k]
                    for q in range(D // 16):
                        sl = pl.ds(q * 16, 16)
                        obuf[s][r0 + k, sl] = (
                            rows[s][r0 + k, sl] + stab_v[sid, sl])

            scatter_desc(g, s).start()

            sp = (s + 2) % NB

            @pl.when(g >= 2)
            def _():
                scatter_desc(g - 2, sp).wait()

            @pl.when(g + 2 < NG)
            def _():
                gather_desc(g + 2, sp).start()

    for g in (NG - 2, NG - 1):
        scatter_desc(g, g % NB).wait()


@jax.jit
def _sc_embed(wids, sids, wtab, stab):
    kern = pl.kernel(
        _body,
        out_type=jax.ShapeDtypeStruct((N, D), jnp.float32),
        mesh=plsc.VectorSubcoreMesh(core_axis_name="c", subcore_axis_name="s"),
        compiler_params=pltpu.CompilerParams(use_tc_tiling_on_sc=False,
                                             needs_layout_passes=False),
        scratch_types=[
            pltpu.VMEM((NG, G), jnp.int32),
            pltpu.VMEM((NG, G), jnp.int32),
            pltpu.VMEM((BINS, D), jnp.float32),
            pltpu.VMEM((G, D), jnp.float32),
            pltpu.VMEM((G, D), jnp.float32),
            pltpu.VMEM((G, D), jnp.float32),
            pltpu.VMEM((G, D), jnp.float32),
            pltpu.VMEM((G, D), jnp.float32),
            pltpu.VMEM((G, D), jnp.float32),
            pltpu.VMEM((G, D), jnp.float32),
            pltpu.VMEM((G, D), jnp.float32),
            pltpu.SemaphoreType.DMA((NB,)),
            pltpu.SemaphoreType.DMA((NB,)),
        ],
    )
    return kern(wids, sids, wtab, stab)


def kernel(input_ids, scores_ids, words_emb, scores_emb):
    wids = input_ids.reshape(NW, NG, G).astype(jnp.int32)
    sids = scores_ids.reshape(NW, NG, G).astype(jnp.int32)
    out = _sc_embed(wids, sids, words_emb, scores_emb)
    return out.reshape(input_ids.shape + (D,))
